# baseline (speedup 1.0000x reference)
import functools
import jax
import jax.numpy as jnp
from jax import lax
from jax.experimental import pallas as pl
from jax.experimental.pallas import tpu as pltpu
from jax.experimental.pallas import tpu_sc as plsc

NC = 2
NS = 16
LANES = 16
NW = NC * NS


def _sc_pool_gather(enc, add0, add1, add2, emb, cat0, cat1, cat2):
  B, L = enc.shape
  V, D = emb.shape
  rows_per_w = B // NW
  groups = rows_per_w // LANES
  assert rows_per_w % LANES == 0

  mesh = plsc.VectorSubcoreMesh(
      core_axis_name="c", subcore_axis_name="s", num_cores=NC,
      num_subcores=NS)

  @functools.partial(
      pl.kernel,
      compiler_params=pltpu.CompilerParams(
          use_tc_tiling_on_sc=False, needs_layout_passes=False),
      out_type=(
          jax.ShapeDtypeStruct((B, D), jnp.float32),
          jax.ShapeDtypeStruct((B,), jnp.float32),
          jax.ShapeDtypeStruct((3, B, D), jnp.float32)
      ),
      mesh=mesh,
      scratch_types=[
          pltpu.VMEM((LANES, L), jnp.int32),
          pltpu.VMEM((LANES * L, D), jnp.float32),
          pltpu.VMEM((3, LANES), jnp.int32),
          pltpu.VMEM((3, LANES, D), jnp.float32),
          pltpu.VMEM((D, LANES), jnp.float32),
          pltpu.VMEM((LANES,), jnp.float32),
          pltpu.VMEM((LANES, D), jnp.float32),
          pltpu.SemaphoreType.DMA,
          pltpu.SemaphoreType.DMA,
      ],
  )
  def sc_fn(enc_ref, a0_ref, a1_ref, a2_ref, emb_ref, c0_ref, c1_ref, c2_ref,
            pooled_out, cnt_out, cat_out,
            idx_v, rows_v, cidxT_v, crows_v, pooled_v, cnt_v,
            outp_v, sem, sem2):
    wid = lax.axis_index("s") * NC + lax.axis_index("c")
    lanes_iota = lax.iota(jnp.int32, LANES)
    zeros16 = jnp.zeros((LANES,), jnp.float32)
    base_rows = lanes_iota * L
    cat_refs = (c0_ref, c1_ref, c2_ref)

    def group_body(g, carry):
      b0 = (wid * groups + g) * LANES
      pltpu.sync_copy(enc_ref.at[pl.ds(b0, LANES), :], idx_v)
      for i, a_ref in enumerate((a0_ref, a1_ref, a2_ref)):
        pltpu.sync_copy(a_ref.at[pl.ds(b0, LANES)], cidxT_v.at[i])
      copies = [
          pltpu.async_copy(emb_ref.at[idx_v.at[j]],
                           rows_v.at[pl.ds(j * L, L), :], sem)
          for j in range(LANES)
      ]
      cat_copies = [
          pltpu.async_copy(cat_refs[i].at[cidxT_v.at[i]], crows_v.at[i],
                           sem2)
          for i in range(3)
      ]
      for d in range(D):
        pooled_v[d] = zeros16
      cnt_v[...] = zeros16
      for c in copies:
        c.wait()

      def token_body(t, tc):
        ridx = base_rows + t
        s0 = zeros16
        s1 = zeros16
        s2 = zeros16
        s3 = zeros16
        for d in range(D):
          v = plsc.load_gather(
              rows_v, [ridx, jnp.full((LANES,), d, jnp.int32)])
          plsc.addupdate(pooled_v.at[d], v)
          if d % 4 == 0:
            s0 = s0 + v
          elif d % 4 == 1:
            s1 = s1 + v
          elif d % 4 == 2:
            s2 = s2 + v
          else:
            s3 = s3 + v
        s = (s0 + s1) + (s2 + s3)
        plsc.addupdate(
            cnt_v.at[:],
            jnp.where(s != 0.0, jnp.float32(1.0), jnp.float32(0.0)))
        return tc

      lax.fori_loop(0, L, token_body, 0)

      for d in range(D):
        plsc.store_scatter(
            outp_v, [lanes_iota, jnp.full((LANES,), d, jnp.int32)],
            pooled_v[d])
      pltpu.sync_copy(outp_v, pooled_out.at[pl.ds(b0, LANES), :])
      pltpu.sync_copy(cnt_v, cnt_out.at[pl.ds(b0, LANES)])
      for c in cat_copies:
        c.wait()
      for i in range(3):
        pltpu.sync_copy(crows_v.at[i], cat_out.at[i, pl.ds(b0, LANES), :])
      return carry

    lax.fori_loop(0, groups, group_body, 0)

  return sc_fn(enc, add0, add1, add2, emb, cat0, cat1, cat2)


def _tc_finalize_matmul(pooled, cnt, cats, fc_w, fc_b):
  B, D = pooled.shape
  C = fc_w.shape[1]
  BB = 512
  cnt2 = cnt.reshape(B, 1)
  fb2 = fc_b.reshape(1, C)

  def tc_body(p_ref, c_ref, cat_ref, w_ref, b_ref, o_ref):
    q = p_ref[...] / c_ref[...]
    q = jnp.where(q != q, jnp.float32(0.0), q)
    big = jnp.float32(3.4028234663852886e38)
    q = jnp.where(q == jnp.inf, big, q)
    q = jnp.where(q == -jnp.inf, -big, q)
    x = q + cat_ref[0] + cat_ref[1] + cat_ref[2]
    o_ref[...] = (
        jnp.dot(x, w_ref[...], preferred_element_type=jnp.float32)
        + b_ref[...])

  return pl.pallas_call(
      tc_body,
      grid=(B // BB,),
      in_specs=[
          pl.BlockSpec((BB, D), lambda i: (i, 0)),
          pl.BlockSpec((BB, 1), lambda i: (i, 0)),
          pl.BlockSpec((3, BB, D), lambda i: (0, i, 0)),
          pl.BlockSpec((D, C), lambda i: (0, 0)),
          pl.BlockSpec((1, C), lambda i: (0, 0)),
      ],
      out_specs=pl.BlockSpec((BB, C), lambda i: (i, 0)),
      out_shape=jax.ShapeDtypeStruct((B, C), jnp.float32),
  )(pooled, cnt2, cats, fc_w, fb2)


def kernel(encoded_text, additional_inputs, emb_table, cat_emb0, cat_emb1,
           cat_emb2, fc_w, fc_b):
  add_i = additional_inputs.astype(jnp.int32)
  pooled, cnt, cats = _sc_pool_gather(
      encoded_text.astype(jnp.int32), add_i[:, 0], add_i[:, 1], add_i[:, 2],
      emb_table, cat_emb0, cat_emb1, cat_emb2)
  return _tc_finalize_matmul(pooled, cnt, cats, fc_w, fc_b)
